# baseline (speedup 1.0000x reference)
import functools
import math

import jax
import jax.numpy as jnp
from jax import lax
from jax.experimental import pallas as pl
from jax.experimental.pallas import tpu as pltpu
from jax.experimental.pallas import tpu_sc as plsc

_D = 128
_SCALE = math.sqrt(float(_D))


def _pack_body(x_ref, o_ref):
    x = x_ref[...] * _SCALE
    n, d = x.shape
    x4 = x.reshape(n, d // 32, 2, 16)
    lo = jax.lax.bitcast_convert_type(x4[:, :, 0, :].astype(jnp.bfloat16),
                                      jnp.uint16).astype(jnp.uint32)
    hi = jax.lax.bitcast_convert_type(x4[:, :, 1, :].astype(jnp.bfloat16),
                                      jnp.uint16).astype(jnp.uint32)
    w = jax.lax.bitcast_convert_type(lo | (hi << 16), jnp.int32)
    o_ref[...] = w.reshape(n, d // 2)


@functools.lru_cache(maxsize=None)
def _make_pack(V, D):
    blk = 2000
    assert V % blk == 0
    return pl.pallas_call(
        _pack_body,
        out_shape=jax.ShapeDtypeStruct((V, D // 2), jnp.int32),
        grid=(V // blk,),
        in_specs=[pl.BlockSpec((blk, D), lambda i: (i, 0))],
        out_specs=pl.BlockSpec((blk, D // 2), lambda i: (i, 0)),
    )


@functools.lru_cache(maxsize=None)
def _make_gather(V, D, B):
    info = plsc.get_sparse_core_info()
    NC, NS = info.num_cores, info.num_subcores
    NW = NC * NS
    C = 128
    G = 2
    CB = C * G
    D2 = D // 2
    assert B % (NW * CB) == 0
    b_per_w = B // NW
    n_idx = b_per_w // C
    n_chunks = b_per_w // CB
    mesh = plsc.VectorSubcoreMesh(core_axis_name="c", subcore_axis_name="s")

    @functools.partial(
        pl.kernel,
        out_type=jax.ShapeDtypeStruct((B, D), jnp.float32),
        mesh=mesh,
        compiler_params=pltpu.CompilerParams(use_tc_tiling_on_sc=False),
        scratch_types=[
            pltpu.VMEM((n_idx, C), jnp.int32),
            pltpu.VMEM((CB, D2), jnp.int32),
            pltpu.VMEM((CB, D2), jnp.int32),
            pltpu.VMEM((CB, D), jnp.float32),
            pltpu.VMEM((CB, D), jnp.float32),
            pltpu.SemaphoreType.DMA,
            pltpu.SemaphoreType.DMA,
            pltpu.SemaphoreType.DMA,
            pltpu.SemaphoreType.DMA,
        ],
    )
    def gather_kernel(idx_hbm, table_hbm, out_hbm,
                      idx_v, pk0, pk1, fb0, fb1, g0, g1, s0, s1):
        wid = lax.axis_index("s") * NC + lax.axis_index("c")
        base = wid * b_per_w
        pk = (pk0, pk1)
        fb = (fb0, fb1)
        gsem = (g0, g1)
        ssem = (s0, s1)

        pltpu.sync_copy(idx_hbm.at[pl.ds(wid * n_idx, n_idx)], idx_v)

        def gather_start(i, b):
            for g in range(G):
                pltpu.async_copy(table_hbm.at[idx_v.at[i * G + g]],
                                 pk[b].at[pl.ds(g * C, C)], gsem[b])

        def gather_wait(i, b):
            for g in range(G):
                pltpu.make_async_copy(
                    table_hbm.at[idx_v.at[i * G + g]],
                    pk[b].at[pl.ds(g * C, C)], gsem[b]).wait()

        def widen_buf(b):
            def sbody(r, carry):
                for u in range(2):
                    row = 2 * r + u
                    for k in range(D2 // 16):
                        v = pk[b][row, pl.ds(16 * k, 16)]
                        lo = jax.lax.bitcast_convert_type(
                            v << 16, jnp.float32)
                        hi = jax.lax.bitcast_convert_type(
                            v & jnp.int32(-65536), jnp.float32)
                        fb[b][row, pl.ds(32 * k, 16)] = lo
                        fb[b][row, pl.ds(32 * k + 16, 16)] = hi
                return carry
            lax.fori_loop(0, CB // 2, sbody, 0)

        def scatter_start(i, b):
            pltpu.async_copy(
                fb[b], out_hbm.at[pl.ds(base + i * CB, CB)], ssem[b])

        def scatter_wait(i, b):
            pltpu.make_async_copy(
                fb[b], out_hbm.at[pl.ds(base + i * CB, CB)], ssem[b]).wait()

        gather_start(0, 0)
        gather_start(1, 1)
        gather_wait(0, 0)
        widen_buf(0)
        scatter_start(0, 0)
        gather_wait(1, 1)
        widen_buf(1)
        scatter_start(1, 1)

        def body(j, carry):
            i0 = 2 * j
            i1 = i0 + 1
            gather_start(i0, 0)
            gather_start(i1, 1)
            scatter_wait(i0 - 2, 0)
            gather_wait(i0, 0)
            widen_buf(0)
            scatter_start(i0, 0)
            scatter_wait(i1 - 2, 1)
            gather_wait(i1, 1)
            widen_buf(1)
            scatter_start(i1, 1)
            return carry

        lax.fori_loop(1, n_chunks // 2, body, 0)

        scatter_wait(n_chunks - 2, 0)
        scatter_wait(n_chunks - 1, 1)

    return gather_kernel


def kernel(Tokens, table):
    S, T = Tokens.shape
    V, D = table.shape
    B = S * T
    packed = _make_pack(V, D)(table)
    idx2d = Tokens.reshape(B // 128, 128).astype(jnp.int32)
    out = _make_gather(V, D, B)(idx2d, packed)
    return out.reshape(S, T, D)


# speedup vs baseline: 3.8731x; 3.8731x over previous
import functools
import math

import jax
import jax.numpy as jnp
from jax import lax
from jax.experimental import pallas as pl
from jax.experimental.pallas import tpu as pltpu
from jax.experimental.pallas import tpu_sc as plsc

_D = 128
_SCALE = math.sqrt(float(_D))


def _scale_body(x_ref, o_ref):
    o_ref[...] = x_ref[...] * _SCALE


@functools.lru_cache(maxsize=None)
def _make_scale(V, D):
    blk = 2000
    assert V % blk == 0
    return pl.pallas_call(
        _scale_body,
        out_shape=jax.ShapeDtypeStruct((V, D), jnp.float32),
        grid=(V // blk,),
        in_specs=[pl.BlockSpec((blk, D), lambda i: (i, 0))],
        out_specs=pl.BlockSpec((blk, D), lambda i: (i, 0)),
    )


@functools.lru_cache(maxsize=None)
def _make_gather(V, D, B):
    info = plsc.get_sparse_core_info()
    NC, NS = info.num_cores, info.num_subcores
    NW = NC * NS
    C = 128
    G = 1
    CB = C * G
    NB = 4
    assert B % (NW * CB) == 0
    b_per_w = B // NW
    n_idx = b_per_w // C
    n_chunks = b_per_w // CB
    mesh = plsc.VectorSubcoreMesh(core_axis_name="c", subcore_axis_name="s")

    @functools.partial(
        pl.kernel,
        out_type=jax.ShapeDtypeStruct((B, D), jnp.float32),
        mesh=mesh,
        scratch_types=[
            pltpu.VMEM((n_idx, C), jnp.int32),
            pltpu.VMEM((CB, D), jnp.float32),
            pltpu.VMEM((CB, D), jnp.float32),
            pltpu.VMEM((CB, D), jnp.float32),
            pltpu.VMEM((CB, D), jnp.float32),
            pltpu.SemaphoreType.DMA,
            pltpu.SemaphoreType.DMA,
            pltpu.SemaphoreType.DMA,
            pltpu.SemaphoreType.DMA,
            pltpu.SemaphoreType.DMA,
            pltpu.SemaphoreType.DMA,
            pltpu.SemaphoreType.DMA,
            pltpu.SemaphoreType.DMA,
        ],
    )
    def gather_kernel(idx_hbm, table_hbm, out_hbm,
                      idx_v, rows0, rows1, rows2, rows3,
                      g0, g1, g2, g3, s0, s1, s2, s3):
        wid = lax.axis_index("s") * NC + lax.axis_index("c")
        base = wid * b_per_w
        rows = (rows0, rows1, rows2, rows3)
        gsem = (g0, g1, g2, g3)
        ssem = (s0, s1, s2, s3)

        pltpu.sync_copy(idx_hbm.at[pl.ds(wid * n_idx, n_idx)], idx_v)

        def gather_start(i, b):
            for g in range(G):
                pltpu.async_copy(table_hbm.at[idx_v.at[i * G + g]],
                                 rows[b].at[pl.ds(g * C, C)], gsem[b])

        def gather_wait(i, b):
            for g in range(G):
                pltpu.make_async_copy(
                    table_hbm.at[idx_v.at[i * G + g]],
                    rows[b].at[pl.ds(g * C, C)], gsem[b]).wait()

        def scale_buf(b):
            def sbody(r, carry):
                for u in range(2):
                    for k in range(D // 16):
                        sl = (2 * r + u, pl.ds(16 * k, 16))
                        rows[b][sl] = rows[b][sl] * _SCALE
                return carry
            lax.fori_loop(0, CB // 2, sbody, 0)

        def scatter_start(i, b):
            pltpu.async_copy(
                rows[b], out_hbm.at[pl.ds(base + i * CB, CB)], ssem[b])

        def scatter_wait(i, b):
            pltpu.make_async_copy(
                rows[b], out_hbm.at[pl.ds(base + i * CB, CB)], ssem[b]).wait()

        def consume(i, b):
            gather_wait(i, b)
            scale_buf(b)
            scatter_start(i, b)

        gather_start(0, 0)
        gather_start(1, 1)
        gather_start(2, 2)
        consume(0, 0)
        gather_start(3, 3)
        consume(1, 1)

        def body(jj, carry):
            i = 4 * jj
            for o, b, b2 in ((2, 2, 0), (3, 3, 1), (4, 0, 2), (5, 1, 3)):
                scatter_wait(i + o - 2, b2)
                gather_start(i + o + 2, b2)
                consume(i + o, b)
            return carry

        lax.fori_loop(0, (n_chunks - 4) // 4, body, 0)

        consume(n_chunks - 2, 2)
        consume(n_chunks - 1, 3)
        scatter_wait(n_chunks - 4, 0)
        scatter_wait(n_chunks - 3, 1)
        scatter_wait(n_chunks - 2, 2)
        scatter_wait(n_chunks - 1, 3)

    return gather_kernel


def kernel(Tokens, table):
    S, T = Tokens.shape
    V, D = table.shape
    B = S * T
    idx2d = Tokens.reshape(B // 128, 128).astype(jnp.int32)
    out = _make_gather(V, D, B)(idx2d, table)
    return out.reshape(S, T, D)
